# Initial kernel scaffold; baseline (speedup 1.0000x reference)
import jax
import jax.numpy as jnp
from jax.experimental import pallas as pl


def kernel(X, prototypes, W):
    raise NotImplementedError("write your pallas kernel here")


# speedup vs baseline: 4.6145x; 4.6145x over previous
import functools

import jax
import jax.numpy as jnp
from jax import lax
from jax.experimental import pallas as pl
from jax.experimental.pallas import tpu as pltpu
from jax.experimental.pallas import tpu_sc as plsc

N, D, M = 16384, 512, 512
C = 16
K = 16
S = D // C
L = 16

NC, NS = 2, 16
NW = NC * NS
ROWS_PER_W = N // NW
SAMP = 8
NCHUNK = ROWS_PER_W // SAMP

ENC_B = 1024


def _encode_body(x_ref, p_ref, c_ref, sx_ref):
    i = pl.program_id(0)
    xb = x_ref[...]
    P = p_ref[...]
    xp = jnp.dot(xb, P, preferred_element_type=jnp.float32)
    psq = jnp.sum(P * P, axis=0)
    d = psq[None, :] - 2.0 * xp
    d3 = d.reshape(ENC_B, C, K)
    am = jnp.argmin(d3, axis=-1).astype(jnp.int32)
    c_ref[...] = am + K * lax.broadcasted_iota(jnp.int32, (ENC_B, C), 1)

    @pl.when(i == 0)
    def _():
        sx_ref[0, 0] = 0.0

    sx_ref[0, 0] += jnp.sum(xb)


def _lut_body(pt_ref, w_ref, sx_ref, lut_ref):
    lut = jnp.dot(pt_ref[...], w_ref[...], preferred_element_type=jnp.float32)
    lut_ref[...] = lut * (sx_ref[0, 0] / float(N * D))


def _agg_body(lut_hbm, codes_hbm, y_hbm, idx_v, stg_v, out_v, sem):
    wid = lax.axis_index("s") * NC + lax.axis_index("c")
    base = wid * ROWS_PER_W

    def chunk(t, _):
        samp0 = base + t * SAMP
        pltpu.sync_copy(codes_hbm.at[pl.ds(samp0 * C, SAMP * C)], idx_v)
        pltpu.async_copy(lut_hbm.at[idx_v], stg_v, sem).wait()

        def samp(s, _):
            srow = s * C
            for j in range(M // L):
                acc = stg_v[srow, pl.ds(j * L, L)]
                for cc in range(1, C):
                    acc = acc + stg_v[srow + cc, pl.ds(j * L, L)]
                out_v[s, pl.ds(j * L, L)] = acc
            return 0

        lax.fori_loop(0, SAMP, samp, 0)
        pltpu.sync_copy(out_v, y_hbm.at[pl.ds(samp0, SAMP)])
        return 0

    lax.fori_loop(0, NCHUNK, chunk, 0)


def kernel(X, prototypes, W):
    pt = jnp.transpose(prototypes, (0, 2, 1))
    P = (jnp.zeros((C, S, C, K), jnp.float32)
         .at[jnp.arange(C), :, jnp.arange(C), :].set(pt)
         .reshape(D, C * K))

    codes, sumx = pl.pallas_call(
        _encode_body,
        grid=(N // ENC_B,),
        in_specs=[
            pl.BlockSpec((ENC_B, D), lambda i: (i, 0)),
            pl.BlockSpec((D, C * K), lambda i: (0, 0)),
        ],
        out_specs=[
            pl.BlockSpec((ENC_B, C), lambda i: (i, 0)),
            pl.BlockSpec(memory_space=pltpu.SMEM),
        ],
        out_shape=[
            jax.ShapeDtypeStruct((N, C), jnp.int32),
            jax.ShapeDtypeStruct((1, 1), jnp.float32),
        ],
        compiler_params=pltpu.CompilerParams(
            dimension_semantics=("arbitrary",)),
    )(X, P)

    luts = pl.pallas_call(
        _lut_body,
        in_specs=[
            pl.BlockSpec((C * K, D), lambda: (0, 0)),
            pl.BlockSpec((D, M), lambda: (0, 0)),
            pl.BlockSpec(memory_space=pltpu.SMEM),
        ],
        out_specs=pl.BlockSpec((C * K, M), lambda: (0, 0)),
        out_shape=jax.ShapeDtypeStruct((C * K, M), jnp.float32),
    )(P.T, W, sumx)

    agg = functools.partial(
        pl.kernel,
        out_type=jax.ShapeDtypeStruct((N, M), jnp.float32),
        mesh=plsc.VectorSubcoreMesh(
            core_axis_name="c", subcore_axis_name="s",
            num_cores=NC, num_subcores=NS),
        scratch_types=[
            pltpu.VMEM((SAMP * C,), jnp.int32),
            pltpu.VMEM((SAMP * C, M), jnp.float32),
            pltpu.VMEM((SAMP, M), jnp.float32),
            pltpu.SemaphoreType.DMA,
        ],
    )(_agg_body)

    return agg(luts, codes.reshape(N * C))


# speedup vs baseline: 11.2511x; 2.4382x over previous
import functools

import jax
import jax.numpy as jnp
from jax import lax
from jax.experimental import pallas as pl
from jax.experimental.pallas import tpu as pltpu
from jax.experimental.pallas import tpu_sc as plsc

N, D, M = 16384, 512, 512
C = 16
K = 16
S = D // C
L = 16
J = C // 2
TROWS = J * K * K

NC, NS = 2, 16
NW = NC * NS
ROWS_PER_W = N // NW
SAMP = 8
GI = SAMP * J
NCHUNK = ROWS_PER_W // SAMP

ENC_B = 1024


def _encode_body(x_ref, p_ref, c_ref, sx_ref):
    i = pl.program_id(0)
    xb = x_ref[...]
    P = p_ref[...]
    xp = jnp.dot(xb, P, preferred_element_type=jnp.float32)
    psq = jnp.sum(P * P, axis=0)
    d = psq[None, :] - 2.0 * xp
    d3 = d.reshape(ENC_B, C, K)
    am = jnp.argmin(d3, axis=-1).astype(jnp.float32)
    lane = lax.broadcasted_iota(jnp.int32, (ENC_B, C), 1)
    wgt = jnp.where(lane % 2 == 0, 16.0, 1.0)
    il = lax.broadcasted_iota(jnp.int32, (C, J), 0)
    ij = lax.broadcasted_iota(jnp.int32, (C, J), 1)
    pairmat = ((il // 2) == ij).astype(jnp.float32)
    code2 = jnp.dot(am * wgt, pairmat,
                    preferred_element_type=jnp.float32)
    c_ref[...] = (code2.astype(jnp.int32)
                  + 256 * lax.broadcasted_iota(jnp.int32, (ENC_B, J), 1))

    @pl.when(i == 0)
    def _():
        sx_ref[0, 0] = 0.0

    sx_ref[0, 0] += jnp.sum(xb)


def _lut_body(pt_ref, w_ref, sx_ref, lut2_ref):
    lut = jnp.dot(pt_ref[...], w_ref[...], preferred_element_type=jnp.float32)
    lut = lut * (sx_ref[0, 0] / float(N * D))
    lr = lut.reshape(J, 2, K, M)
    la = lr[:, 0]
    lb = lr[:, 1]
    lut2 = la[:, :, None, :] + lb[:, None, :, :]
    lut2_ref[...] = lut2.reshape(TROWS, M)


def _agg_body(lut_hbm, codes_hbm, y_hbm, idx_all, stg, outb, semg, semo):
    wid = lax.axis_index("s") * NC + lax.axis_index("c")
    base = wid * ROWS_PER_W
    pltpu.sync_copy(codes_hbm.at[pl.ds(base * J, ROWS_PER_W * J)], idx_all)
    pltpu.async_copy(lut_hbm.at[idx_all.at[pl.ds(0, GI)]], stg[0], semg[0])

    def reduce_chunk(ch, p):
        nxt = ch + 1

        @pl.when(nxt < NCHUNK)
        def _():
            pltpu.async_copy(lut_hbm.at[idx_all.at[pl.ds(nxt * GI, GI)]],
                             stg[1 - p], semg[1 - p])

        pltpu.make_async_copy(lut_hbm.at[idx_all.at[pl.ds(0, GI)]],
                              stg[p], semg[p]).wait()

        @pl.when(ch >= 2)
        def _():
            pltpu.make_async_copy(outb[p], y_hbm.at[pl.ds(0, SAMP)],
                                  semo[p]).wait()

        def samp(s, _):
            srow = s * J
            for j in range(M // L):
                acc = stg[p][srow, pl.ds(j * L, L)]
                for cc in range(1, J):
                    acc = acc + stg[p][srow + cc, pl.ds(j * L, L)]
                outb[p][s, pl.ds(j * L, L)] = acc
            return 0

        lax.fori_loop(0, SAMP, samp, 0)
        pltpu.async_copy(outb[p], y_hbm.at[pl.ds(base + ch * SAMP, SAMP)],
                         semo[p])

    def pair_body(i, _):
        reduce_chunk(2 * i, 0)
        reduce_chunk(2 * i + 1, 1)
        return 0

    lax.fori_loop(0, NCHUNK // 2, pair_body, 0)
    for p in range(2):
        pltpu.make_async_copy(outb[p], y_hbm.at[pl.ds(0, SAMP)],
                              semo[p]).wait()


def kernel(X, prototypes, W):
    pt = jnp.transpose(prototypes, (0, 2, 1))
    P = (jnp.zeros((C, S, C, K), jnp.float32)
         .at[jnp.arange(C), :, jnp.arange(C), :].set(pt)
         .reshape(D, C * K))

    codes, sumx = pl.pallas_call(
        _encode_body,
        grid=(N // ENC_B,),
        in_specs=[
            pl.BlockSpec((ENC_B, D), lambda i: (i, 0)),
            pl.BlockSpec((D, C * K), lambda i: (0, 0)),
        ],
        out_specs=[
            pl.BlockSpec((ENC_B, J), lambda i: (i, 0)),
            pl.BlockSpec(memory_space=pltpu.SMEM),
        ],
        out_shape=[
            jax.ShapeDtypeStruct((N, J), jnp.int32),
            jax.ShapeDtypeStruct((1, 1), jnp.float32),
        ],
        compiler_params=pltpu.CompilerParams(
            dimension_semantics=("arbitrary",)),
    )(X, P)

    lut2 = pl.pallas_call(
        _lut_body,
        in_specs=[
            pl.BlockSpec((C * K, D), lambda: (0, 0)),
            pl.BlockSpec((D, M), lambda: (0, 0)),
            pl.BlockSpec(memory_space=pltpu.SMEM),
        ],
        out_specs=pl.BlockSpec((TROWS, M), lambda: (0, 0)),
        out_shape=jax.ShapeDtypeStruct((TROWS, M), jnp.float32),
    )(P.T, W, sumx)

    agg = functools.partial(
        pl.kernel,
        out_type=jax.ShapeDtypeStruct((N, M), jnp.float32),
        mesh=plsc.VectorSubcoreMesh(
            core_axis_name="c", subcore_axis_name="s",
            num_cores=NC, num_subcores=NS),
        scratch_types=[
            pltpu.VMEM((ROWS_PER_W * J,), jnp.int32),
            [pltpu.VMEM((GI, M), jnp.float32) for _ in range(2)],
            [pltpu.VMEM((SAMP, M), jnp.float32) for _ in range(2)],
            [pltpu.SemaphoreType.DMA for _ in range(2)],
            [pltpu.SemaphoreType.DMA for _ in range(2)],
        ],
    )(_agg_body)

    return agg(lut2, codes.reshape(N * J))


# speedup vs baseline: 14.9786x; 1.3313x over previous
import functools

import jax
import jax.numpy as jnp
from jax import lax
from jax.experimental import pallas as pl
from jax.experimental.pallas import tpu as pltpu
from jax.experimental.pallas import tpu_sc as plsc

N, D, M = 16384, 512, 512
C = 16
K = 16
S = D // C
L = 16
J = C // 2
TROWS = J * K * K

NC, NS = 2, 16
NW = NC * NS
ROWS_PER_W = N // NW
SAMP = 8
GI = SAMP * J
NCHUNK = ROWS_PER_W // SAMP

ENC_B = 1024


def _encode_body(x_ref, p_ref, pt_ref, w_ref, c_ref, lut2_ref, sx_ref):
    i = pl.program_id(0)
    xb = x_ref[...]
    P = p_ref[...]
    xp_t = lax.dot_general(P, xb, (((0,), (1,)), ((), ())),
                           preferred_element_type=jnp.float32)
    psq = jnp.sum(P * P, axis=0)
    d3 = (psq[:, None] - 2.0 * xp_t).reshape(C, K, ENC_B)
    m = jnp.min(d3, axis=1)
    ik = lax.broadcasted_iota(jnp.int32, (C, K, ENC_B), 1)
    am = jnp.min(jnp.where(d3 == m[:, None, :], ik, K), axis=1)
    am4 = am.reshape(J, 2, ENC_B)
    code2_t = (16 * am4[:, 0] + am4[:, 1]
               + 256 * lax.broadcasted_iota(jnp.int32, (J, ENC_B), 0))
    c_ref[...] = code2_t.T

    @pl.when(i == 0)
    def _():
        sx_ref[0, 0] = 0.0

    sx_ref[0, 0] += jnp.sum(xb)

    @pl.when(i == pl.num_programs(0) - 1)
    def _():
        lut = jnp.dot(pt_ref[...], w_ref[...],
                      preferred_element_type=jnp.float32)
        lut = lut * (sx_ref[0, 0] / float(N * D))
        lr = lut.reshape(J, 2, K, M)
        la = lr[:, 0]
        lb = lr[:, 1]
        lut2 = la[:, :, None, :] + lb[:, None, :, :]
        lut2_ref[...] = lut2.reshape(TROWS, M)


def _agg_body(lut_hbm, codes_hbm, y_hbm, idx_all, stg, outb, semg, semo):
    wid = lax.axis_index("s") * NC + lax.axis_index("c")
    base = wid * ROWS_PER_W
    pltpu.sync_copy(codes_hbm.at[pl.ds(base * J, ROWS_PER_W * J)], idx_all)
    pltpu.async_copy(lut_hbm.at[idx_all.at[pl.ds(0, GI)]], stg[0], semg[0])

    def reduce_chunk(ch, p):
        nxt = ch + 1

        @pl.when(nxt < NCHUNK)
        def _():
            pltpu.async_copy(lut_hbm.at[idx_all.at[pl.ds(nxt * GI, GI)]],
                             stg[1 - p], semg[1 - p])

        pltpu.make_async_copy(lut_hbm.at[idx_all.at[pl.ds(0, GI)]],
                              stg[p], semg[p]).wait()

        @pl.when(ch >= 2)
        def _():
            pltpu.make_async_copy(outb[p], y_hbm.at[pl.ds(0, SAMP)],
                                  semo[p]).wait()

        def samp(s, _):
            srow = s * J
            for j in range(M // L):
                acc = stg[p][srow, pl.ds(j * L, L)]
                for cc in range(1, J):
                    acc = acc + stg[p][srow + cc, pl.ds(j * L, L)]
                outb[p][s, pl.ds(j * L, L)] = acc
            return 0

        lax.fori_loop(0, SAMP, samp, 0)
        pltpu.async_copy(outb[p], y_hbm.at[pl.ds(base + ch * SAMP, SAMP)],
                         semo[p])

    def pair_body(i, _):
        reduce_chunk(2 * i, 0)
        reduce_chunk(2 * i + 1, 1)
        return 0

    lax.fori_loop(0, NCHUNK // 2, pair_body, 0)
    for p in range(2):
        pltpu.make_async_copy(outb[p], y_hbm.at[pl.ds(0, SAMP)],
                              semo[p]).wait()


def kernel(X, prototypes, W):
    pt = jnp.transpose(prototypes, (0, 2, 1))
    P = (jnp.zeros((C, S, C, K), jnp.float32)
         .at[jnp.arange(C), :, jnp.arange(C), :].set(pt)
         .reshape(D, C * K))

    codes, lut2, sumx = pl.pallas_call(
        _encode_body,
        grid=(N // ENC_B,),
        in_specs=[
            pl.BlockSpec((ENC_B, D), lambda i: (i, 0)),
            pl.BlockSpec((D, C * K), lambda i: (0, 0)),
            pl.BlockSpec((C * K, D), lambda i: (0, 0)),
            pl.BlockSpec((D, M), lambda i: (0, 0)),
        ],
        out_specs=[
            pl.BlockSpec((ENC_B, J), lambda i: (i, 0)),
            pl.BlockSpec((TROWS, M), lambda i: (0, 0)),
            pl.BlockSpec(memory_space=pltpu.SMEM),
        ],
        out_shape=[
            jax.ShapeDtypeStruct((N, J), jnp.int32),
            jax.ShapeDtypeStruct((TROWS, M), jnp.float32),
            jax.ShapeDtypeStruct((1, 1), jnp.float32),
        ],
        compiler_params=pltpu.CompilerParams(
            dimension_semantics=("arbitrary",)),
    )(X, P, P.T, W)

    agg = functools.partial(
        pl.kernel,
        out_type=jax.ShapeDtypeStruct((N, M), jnp.float32),
        mesh=plsc.VectorSubcoreMesh(
            core_axis_name="c", subcore_axis_name="s",
            num_cores=NC, num_subcores=NS),
        scratch_types=[
            pltpu.VMEM((ROWS_PER_W * J,), jnp.int32),
            [pltpu.VMEM((GI, M), jnp.float32) for _ in range(2)],
            [pltpu.VMEM((SAMP, M), jnp.float32) for _ in range(2)],
            [pltpu.SemaphoreType.DMA for _ in range(2)],
            [pltpu.SemaphoreType.DMA for _ in range(2)],
        ],
    )(_agg_body)

    return agg(lut2, codes.reshape(N * J))
